# Initial kernel scaffold; baseline (speedup 1.0000x reference)
import jax
import jax.numpy as jnp
from jax.experimental import pallas as pl


def kernel(x, edge_index, batch, eps, Ws1, bs1, Ws2, bs2, gammas, betas, fc1_w, fc1_b, fc2_w, fc2_b, fc3_w, fc3_b):
    raise NotImplementedError("write your pallas kernel here")


# speedup vs baseline: 4.3723x; 4.3723x over previous
import functools

import jax
import jax.numpy as jnp
from jax import lax
from jax.experimental import pallas as pl
from jax.experimental.pallas import tpu as pltpu
from jax.experimental.pallas import tpu_sc as plsc

N = 10000
E = 320000
D = 128
G = 64
C = 10
L = 4

NC = 2
NS = 16
NW = NC * NS
EPW = E // NW
CH = 80
NCHUNK = EPW // CH
CZ = 200
NCZ = N // CZ

BLK = 2000
NBLK = N // BLK


def _sc_agg_body(src_hbm, dst_hbm, h_hbm, zero_hbm, out_hbm,
                 sidx, didx, rows, agg_sh, sem):
    c = lax.axis_index("c")
    s = lax.axis_index("s")
    wid = c * NS + s

    def _zero(cid):
        @pl.when(cid < NCZ)
        def _():
            pltpu.sync_copy(zero_hbm, agg_sh.at[pl.ds(cid * CZ, CZ)])

    for j in range((NCZ + NS - 1) // NS):
        _zero(s + j * NS)
    plsc.subcore_barrier()

    def body(i, carry):
        base = wid * EPW + i * CH
        pltpu.sync_copy(src_hbm.at[pl.ds(base, CH)], sidx)
        pltpu.sync_copy(dst_hbm.at[pl.ds(base, CH)], didx)
        pltpu.async_copy(h_hbm.at[sidx], rows, sem).wait()
        pltpu.sync_copy(rows, agg_sh.at[didx], add=True)
        return carry

    lax.fori_loop(0, NCHUNK, body, 0)
    plsc.subcore_barrier()

    def _writeout(cid):
        @pl.when(cid < NCZ)
        def _():
            pltpu.sync_copy(agg_sh.at[pl.ds(cid * CZ, CZ)],
                            out_hbm.at[pl.ds(c * N + cid * CZ, CZ)])

    for j in range((NCZ + NS - 1) // NS):
        _writeout(s + j * NS)


@functools.lru_cache(maxsize=None)
def _get_sc_agg():
    return pl.kernel(
        _sc_agg_body,
        out_type=jax.ShapeDtypeStruct((NC * N, D), jnp.float32),
        mesh=plsc.VectorSubcoreMesh(
            core_axis_name="c", subcore_axis_name="s",
            num_cores=NC, num_subcores=NS),
        scratch_types=[
            pltpu.VMEM((CH,), jnp.int32),
            pltpu.VMEM((CH,), jnp.int32),
            pltpu.VMEM((CH, D), jnp.float32),
            pltpu.VMEM_SHARED((N, D), jnp.float32),
            pltpu.SemaphoreType.DMA,
        ],
    )


def _sc_agg(src, dst, h, zero_block):
    return _get_sc_agg()(src, dst, h, zero_block)


def _std_body(x_ref, o_ref, sum_ref, sq_ref):
    k = pl.program_id(0)
    j = pl.program_id(1)

    @pl.when(jnp.logical_and(k == 0, j == 0))
    def _():
        sum_ref[...] = jnp.zeros_like(sum_ref)
        sq_ref[...] = jnp.zeros_like(sq_ref)

    @pl.when(k == 0)
    def _():
        xb = x_ref[...]
        sum_ref[...] += jnp.sum(xb, axis=0, keepdims=True)
        sq_ref[...] += jnp.sum(xb * xb, axis=0, keepdims=True)

    @pl.when(k == 1)
    def _():
        mu = sum_ref[...] / N
        var = sq_ref[...] / N - mu * mu
        sd = jnp.sqrt(jnp.maximum(var, 0.0))
        sd = jnp.where(sd == 0.0, 1.0, sd)
        o_ref[...] = (x_ref[...] - mu) / sd


def _standardize(x):
    return pl.pallas_call(
        _std_body,
        grid=(2, NBLK),
        in_specs=[pl.BlockSpec((BLK, D), lambda k, j: (j, 0))],
        out_specs=pl.BlockSpec((BLK, D), lambda k, j: (j, 0)),
        out_shape=jax.ShapeDtypeStruct((N, D), jnp.float32),
        scratch_shapes=[
            pltpu.VMEM((1, D), jnp.float32),
            pltpu.VMEM((1, D), jnp.float32),
        ],
    )(x)


def _mlp_body(h_ref, a0_ref, a1_ref, eps_ref, w1_ref, b1_ref, w2_ref, b2_ref,
              gm_ref, bt_ref, o_ref, z2_ref, sum_ref, sq_ref):
    k = pl.program_id(0)
    j = pl.program_id(1)

    @pl.when(jnp.logical_and(k == 0, j == 0))
    def _():
        sum_ref[...] = jnp.zeros_like(sum_ref)
        sq_ref[...] = jnp.zeros_like(sq_ref)

    @pl.when(k == 0)
    def _():
        u = eps_ref[...] * h_ref[...] + a0_ref[...] + a1_ref[...]
        z1 = jnp.maximum(
            jnp.dot(u, w1_ref[...], preferred_element_type=jnp.float32)
            + b1_ref[...], 0.0)
        z2 = jnp.maximum(
            jnp.dot(z1, w2_ref[...], preferred_element_type=jnp.float32)
            + b2_ref[...], 0.0)
        z2_ref[pl.ds(j * BLK, BLK), :] = z2
        sum_ref[...] += jnp.sum(z2, axis=0, keepdims=True)
        sq_ref[...] += jnp.sum(z2 * z2, axis=0, keepdims=True)

    @pl.when(k == 1)
    def _():
        mu = sum_ref[...] / N
        var = sq_ref[...] / N - mu * mu
        inv = gm_ref[...] * lax.rsqrt(jnp.maximum(var, 0.0) + 1e-5)
        z2 = z2_ref[pl.ds(j * BLK, BLK), :]
        o_ref[...] = jnp.maximum((z2 - mu) * inv + bt_ref[...], 0.0)


def _gin_mlp(h, agg2, eps_i, w1, b1, w2, b2, gamma, beta):
    one_eps = jnp.reshape(1.0 + eps_i, (1, 1))
    return pl.pallas_call(
        _mlp_body,
        grid=(2, NBLK),
        in_specs=[
            pl.BlockSpec((BLK, D), lambda k, j: (j, 0)),
            pl.BlockSpec((BLK, D), lambda k, j: (j, 0)),
            pl.BlockSpec((BLK, D), lambda k, j: (j + NBLK, 0)),
            pl.BlockSpec((1, 1), lambda k, j: (0, 0)),
            pl.BlockSpec((D, D), lambda k, j: (0, 0)),
            pl.BlockSpec((1, D), lambda k, j: (0, 0)),
            pl.BlockSpec((D, D), lambda k, j: (0, 0)),
            pl.BlockSpec((1, D), lambda k, j: (0, 0)),
            pl.BlockSpec((1, D), lambda k, j: (0, 0)),
            pl.BlockSpec((1, D), lambda k, j: (0, 0)),
        ],
        out_specs=pl.BlockSpec((BLK, D), lambda k, j: (j, 0)),
        out_shape=jax.ShapeDtypeStruct((N, D), jnp.float32),
        scratch_shapes=[
            pltpu.VMEM((N, D), jnp.float32),
            pltpu.VMEM((1, D), jnp.float32),
            pltpu.VMEM((1, D), jnp.float32),
        ],
    )(h, agg2, agg2, one_eps, w1, jnp.reshape(b1, (1, D)), w2,
      jnp.reshape(b2, (1, D)), jnp.reshape(gamma, (1, D)),
      jnp.reshape(beta, (1, D)))


def _head_body(h_ref, b_ref, f1w_ref, f1b_ref, f2w_ref, f2b_ref, f3w_ref,
               f3b_ref, o_ref, pool_ref):
    j = pl.program_id(0)

    @pl.when(j == 0)
    def _():
        pool_ref[...] = jnp.zeros_like(pool_ref)

    bids = b_ref[0, 0, :]
    gid = lax.broadcasted_iota(jnp.int32, (BLK, G), 1)
    oh = (bids[:, None] == gid).astype(jnp.float32)
    pool_ref[...] += lax.dot_general(
        oh, h_ref[...], (((0,), (0,)), ((), ())),
        preferred_element_type=jnp.float32)

    @pl.when(j == NBLK - 1)
    def _():
        z = jnp.maximum(
            jnp.dot(pool_ref[...], f1w_ref[...],
                    preferred_element_type=jnp.float32) + f1b_ref[...], 0.0)
        z = jnp.maximum(
            jnp.dot(z, f2w_ref[...],
                    preferred_element_type=jnp.float32) + f2b_ref[...], 0.0)
        o_ref[...] = (jnp.dot(z, f3w_ref[...],
                              preferred_element_type=jnp.float32)
                      + f3b_ref[...])


def _pool_head(h, batch, f1w, f1b, f2w, f2b, f3w, f3b):
    b3 = jnp.reshape(batch, (NBLK, 1, BLK))
    return pl.pallas_call(
        _head_body,
        grid=(NBLK,),
        in_specs=[
            pl.BlockSpec((BLK, D), lambda j: (j, 0)),
            pl.BlockSpec((1, 1, BLK), lambda j: (j, 0, 0)),
            pl.BlockSpec((D, D), lambda j: (0, 0)),
            pl.BlockSpec((1, D), lambda j: (0, 0)),
            pl.BlockSpec((D, D // 2), lambda j: (0, 0)),
            pl.BlockSpec((1, D // 2), lambda j: (0, 0)),
            pl.BlockSpec((D // 2, C), lambda j: (0, 0)),
            pl.BlockSpec((1, C), lambda j: (0, 0)),
        ],
        out_specs=pl.BlockSpec((G, C), lambda j: (0, 0)),
        out_shape=jax.ShapeDtypeStruct((G, C), jnp.float32),
        scratch_shapes=[pltpu.VMEM((G, D), jnp.float32)],
    )(h, b3, f1w, jnp.reshape(f1b, (1, D)), f2w,
      jnp.reshape(f2b, (1, D // 2)), f3w, jnp.reshape(f3b, (1, C)))


def kernel(x, edge_index, batch, eps, Ws1, bs1, Ws2, bs2, gammas, betas,
           fc1_w, fc1_b, fc2_w, fc2_b, fc3_w, fc3_b):
    src = edge_index[0]
    dst = edge_index[1]
    zero_block = jnp.zeros((CZ, D), jnp.float32)

    h = _standardize(x)
    for i in range(L):
        agg2 = _sc_agg(src, dst, h, zero_block)
        h = _gin_mlp(h, agg2, eps[i], Ws1[i], bs1[i], Ws2[i], bs2[i],
                     gammas[i], betas[i])
    return _pool_head(h, batch, fc1_w, fc1_b, fc2_w, fc2_b, fc3_w, fc3_b)
